# baseline (speedup 1.0000x reference)
import functools

import jax
import jax.numpy as jnp
from jax.experimental import pallas as pl


def _node_attention_block(adj_ref, emb_ref, hv_ref, out_ref):
    emb = emb_ref[:]
    score = jnp.dot(emb, hv_ref[:],
                    preferred_element_type=jnp.float32)
    w = jnp.exp(score - jnp.max(score))
    rhs = jnp.concatenate([emb * w, w], axis=1)
    acc = jnp.dot(adj_ref[:], rhs,
                  preferred_element_type=jnp.float32)
    out_ref[:] = acc[:, :-1] / acc[:, -1:]


@jax.jit
def kernel(emb, adj, H_v):
    n, d = emb.shape
    block_rows = 256
    grid = (n // block_rows,)
    return pl.pallas_call(
        _node_attention_block,
        grid=grid,
        in_specs=[
            pl.BlockSpec((block_rows, n), lambda i: (i, 0)),
            pl.BlockSpec((n, d), lambda i: (0, 0)),
            pl.BlockSpec((d, 1), lambda i: (0, 0)),
        ],
        out_specs=pl.BlockSpec((block_rows, d), lambda i: (i, 0)),
        out_shape=jax.ShapeDtypeStruct((n, d), jnp.float32),
    )(adj, emb, H_v)


# speedup vs baseline: 1.4493x; 1.4493x over previous
import functools

import jax
import jax.numpy as jnp
from jax.experimental import pallas as pl


def _node_attention_block(adj_ref, emb_ref, hv_ref, out_ref):
    emb = emb_ref[:]
    score = jnp.dot(emb, hv_ref[:],
                    preferred_element_type=jnp.float32)
    w = jnp.exp(score - jnp.max(score))
    rhs = jnp.concatenate([emb * w, w], axis=1)
    acc = jnp.dot(adj_ref[:], rhs,
                  preferred_element_type=jnp.float32)
    out_ref[:] = acc[:, :-1] / acc[:, -1:]


@jax.jit
def kernel(emb, adj, H_v):
    n, d = emb.shape
    block_rows = 1024
    grid = (n // block_rows,)
    return pl.pallas_call(
        _node_attention_block,
        grid=grid,
        in_specs=[
            pl.BlockSpec((block_rows, n), lambda i: (i, 0)),
            pl.BlockSpec((n, d), lambda i: (0, 0)),
            pl.BlockSpec((d, 1), lambda i: (0, 0)),
        ],
        out_specs=pl.BlockSpec((block_rows, d), lambda i: (i, 0)),
        out_shape=jax.ShapeDtypeStruct((n, d), jnp.float32),
    )(adj, emb, H_v)
